# Initial kernel scaffold; baseline (speedup 1.0000x reference)
import jax
import jax.numpy as jnp
from jax.experimental import pallas as pl


def kernel(feat, edge_index, l0_w1, l0_b1, l0_w2, l0_b2, l0_aw, l0_ab, l0_aq, l1_w1, l1_b1, l1_w2, l1_b2, l1_aw, l1_ab, l1_aq):
    raise NotImplementedError("write your pallas kernel here")


# speedup vs baseline: 2.9382x; 2.9382x over previous
import functools

import jax
import jax.numpy as jnp
from jax import lax
from jax.experimental import pallas as pl
from jax.experimental.pallas import tpu as pltpu
from jax.experimental.pallas import tpu_sc as plsc

_N = 10000
_R = 2
_E = 160000
_D = 128
_OUT = 128
_DA = 64

_NT = 16
_EPT = _E // _NT
_CH = 128
_EPTP = 10240
_NCH = _EPTP // _CH
_NPH = 2
_NCHP = _NCH // _NPH
_TRASH = _N
_NPAD = 10240
_RPT = _NPAD // _NT

_B = 1000
_NB = _N // _B


def _sc_scatter(h, ei, zeros):
    mesh = plsc.VectorSubcoreMesh(core_axis_name="c", subcore_axis_name="s")

    @functools.partial(
        pl.kernel,
        mesh=mesh,
        out_type=jax.ShapeDtypeStruct((_R, _NPAD, _OUT), jnp.float32),
        scratch_types=[
            pltpu.VMEM((_NCHP, _CH), jnp.int32),
            pltpu.VMEM((_NCHP, _CH), jnp.int32),
            pltpu.VMEM((2, _CH, _OUT), jnp.float32),
            pltpu.VMEM_SHARED((_NPAD, _OUT), jnp.float32),
            pltpu.SemaphoreType.DMA,
            pltpu.SemaphoreType.DMA,
        ],
    )
    def k(h_hbm, ei_hbm, z_hbm, out_hbm, src_v, dst_v, rows_v, agg_sh, sem0, sem1):
        c = lax.axis_index("c")
        s = lax.axis_index("s")
        sems = (sem0, sem1)

        def stage_and_prime(p):
            rr = pl.ds(p * _NCHP, _NCHP)
            pltpu.sync_copy(ei_hbm.at[c, 0, s, rr], src_v)
            pltpu.sync_copy(ei_hbm.at[c, 1, s, rr], dst_v)
            pltpu.async_copy(h_hbm.at[src_v.at[0]], rows_v.at[0], sem0)
            pltpu.async_copy(h_hbm.at[src_v.at[1]], rows_v.at[1], sem1)

        def halfstep(j, b):
            pltpu.make_async_copy(
                h_hbm.at[src_v.at[j]], rows_v.at[b], sems[b]).wait()
            pltpu.sync_copy(rows_v.at[b], agg_sh.at[dst_v.at[j]], add=True)

            @pl.when(j + 2 < _NCHP)
            def _():
                pltpu.async_copy(
                    h_hbm.at[src_v.at[j + 2]], rows_v.at[b], sems[b])

        def body(i, carry):
            halfstep(i * 2, 0)
            halfstep(i * 2 + 1, 1)
            return carry

        stage_and_prime(0)
        rows = pl.ds(s * _RPT, _RPT)
        pltpu.sync_copy(z_hbm.at[rows], agg_sh.at[rows])
        plsc.subcore_barrier()
        lax.fori_loop(0, _NCHP // 2, body, 0)
        for p in range(1, _NPH):
            stage_and_prime(p)
            lax.fori_loop(0, _NCHP // 2, body, 0)
        plsc.subcore_barrier()
        pltpu.sync_copy(agg_sh.at[rows], out_hbm.at[c, rows])

    return k(h, ei, zeros)


def _mlp(x, w1_ref, b1_ref, w2_ref, b2_ref):
    x1 = jnp.maximum(
        jnp.dot(x, w1_ref[...], preferred_element_type=jnp.float32)
        + b1_ref[...], 0.0)
    return jnp.maximum(
        jnp.dot(x1, w2_ref[...], preferred_element_type=jnp.float32)
        + b2_ref[...], 0.0)


_W_SPECS = [
    pl.BlockSpec((_D, _OUT), lambda i: (0, 0)),
    pl.BlockSpec((1, _OUT), lambda i: (0, 0)),
    pl.BlockSpec((_OUT, _OUT), lambda i: (0, 0)),
    pl.BlockSpec((1, _OUT), lambda i: (0, 0)),
    pl.BlockSpec((_OUT, _DA), lambda i: (0, 0)),
    pl.BlockSpec((1, _DA), lambda i: (0, 0)),
    pl.BlockSpec((_DA, 1), lambda i: (0, 0)),
]


def _tc_mid(h, agg, w1, b1, w2, b2, aw, ab, aq):
    def body(h_ref, agg_ref, w1_ref, b1_ref, w2_ref, b2_ref,
             aw_ref, ab_ref, aq_ref, hst_ref, s_ref):
        i = pl.program_id(0)

        @pl.when(i == 0)
        def _():
            s_ref[0] = 0.0
            s_ref[1] = 0.0

        hb = h_ref[...]
        for r in range(_R):
            x2 = _mlp(hb + agg_ref[r], w1_ref, b1_ref, w2_ref, b2_ref)
            hst_ref[r] = x2
            t = jnp.tanh(
                jnp.dot(x2, aw_ref[...], preferred_element_type=jnp.float32)
                + ab_ref[...])
            s_ref[r] += jnp.sum(
                jnp.dot(t, aq_ref[...], preferred_element_type=jnp.float32))

    return pl.pallas_call(
        body,
        grid=(_NB,),
        in_specs=[
            pl.BlockSpec((_B, _D), lambda i: (i, 0)),
            pl.BlockSpec((_R, _B, _OUT), lambda i: (0, i, 0)),
        ] + _W_SPECS,
        out_specs=[
            pl.BlockSpec((_R, _B, _OUT), lambda i: (0, i, 0)),
            pl.BlockSpec(memory_space=pltpu.SMEM),
        ],
        out_shape=[
            jax.ShapeDtypeStruct((_R, _N, _OUT), jnp.float32),
            jax.ShapeDtypeStruct((2,), jnp.float32),
        ],
    )(h, agg, w1, b1.reshape(1, _OUT), w2, b2.reshape(1, _OUT),
      aw, ab.reshape(1, _DA), aq.reshape(_DA, 1))


def _betas(s_ref):
    s0 = s_ref[0] / _N
    s1 = s_ref[1] / _N
    m = jnp.maximum(s0, s1)
    e0 = jnp.exp(s0 - m)
    e1 = jnp.exp(s1 - m)
    b0 = e0 / (e0 + e1)
    return b0, 1.0 - b0


def _tc_combine(hst, ssum):
    def body(s_ref, hst_ref, out_ref):
        b0, b1 = _betas(s_ref)
        out_ref[...] = b0 * hst_ref[0] + b1 * hst_ref[1]

    return pl.pallas_call(
        body,
        grid=(_NB,),
        in_specs=[
            pl.BlockSpec(memory_space=pltpu.SMEM),
            pl.BlockSpec((_R, _B, _OUT), lambda i: (0, i, 0)),
        ],
        out_specs=pl.BlockSpec((_B, _OUT), lambda i: (i, 0)),
        out_shape=jax.ShapeDtypeStruct((_N, _OUT), jnp.float32),
    )(ssum, hst)


def _tc_final(h, agg, w1, b1, w2, b2, aw, ab, aq):
    def body(h_ref, agg_ref, w1_ref, b1_ref, w2_ref, b2_ref,
             aw_ref, ab_ref, aq_ref, out_ref, s_scr, cs_scr):
        i = pl.program_id(0)

        @pl.when(i == 0)
        def _():
            s_scr[0] = 0.0
            s_scr[1] = 0.0
            cs_scr[...] = jnp.zeros_like(cs_scr)

        hb = h_ref[...]
        for r in range(_R):
            x2 = _mlp(hb + agg_ref[r], w1_ref, b1_ref, w2_ref, b2_ref)
            cs_scr[r] += jnp.sum(x2, axis=0, keepdims=True)
            t = jnp.tanh(
                jnp.dot(x2, aw_ref[...], preferred_element_type=jnp.float32)
                + ab_ref[...])
            s_scr[r] += jnp.sum(
                jnp.dot(t, aq_ref[...], preferred_element_type=jnp.float32))

        @pl.when(i == _NB - 1)
        def _():
            b0, b1 = _betas(s_scr)
            out_ref[...] = b0 * cs_scr[0] + b1 * cs_scr[1]

    return pl.pallas_call(
        body,
        grid=(_NB,),
        in_specs=[
            pl.BlockSpec((_B, _OUT), lambda i: (i, 0)),
            pl.BlockSpec((_R, _B, _OUT), lambda i: (0, i, 0)),
        ] + _W_SPECS,
        out_specs=pl.BlockSpec((1, _OUT), lambda i: (0, 0)),
        out_shape=jax.ShapeDtypeStruct((1, _OUT), jnp.float32),
        scratch_shapes=[
            pltpu.SMEM((2,), jnp.float32),
            pltpu.VMEM((_R, 1, _OUT), jnp.float32),
        ],
    )(h, agg, w1, b1.reshape(1, _OUT), w2, b2.reshape(1, _OUT),
      aw, ab.reshape(1, _DA), aq.reshape(_DA, 1))


def kernel(feat, edge_index, l0_w1, l0_b1, l0_w2, l0_b2, l0_aw, l0_ab, l0_aq,
           l1_w1, l1_b1, l1_w2, l1_b2, l1_aw, l1_ab, l1_aq):
    ei32 = edge_index.astype(jnp.int32)
    srcs = jnp.pad(ei32[:, 0].reshape(_R, _NT, _EPT),
                   ((0, 0), (0, 0), (0, _EPTP - _EPT)))
    dsts = jnp.pad(ei32[:, 1].reshape(_R, _NT, _EPT),
                   ((0, 0), (0, 0), (0, _EPTP - _EPT)),
                   constant_values=_TRASH)
    ei = jnp.stack([srcs, dsts], axis=1).reshape(_R, 2, _NT, _NCH, _CH)
    zeros = jnp.zeros((_NPAD, _OUT), jnp.float32)

    agg0 = _sc_scatter(feat, ei, zeros)
    hst0, s0 = _tc_mid(feat, agg0, l0_w1, l0_b1, l0_w2, l0_b2,
                       l0_aw, l0_ab, l0_aq)
    h1 = _tc_combine(hst0, s0)

    agg1 = _sc_scatter(h1, ei, zeros)
    out = _tc_final(h1, agg1, l1_w1, l1_b1, l1_w2, l1_b2,
                    l1_aw, l1_ab, l1_aq)
    return out.reshape(_OUT)
